# Initial kernel scaffold; baseline (speedup 1.0000x reference)
import jax
import jax.numpy as jnp
from jax.experimental import pallas as pl


def kernel(x, w1, wres):
    raise NotImplementedError("write your pallas kernel here")


# speedup vs baseline: 1.1998x; 1.1998x over previous
import functools

import jax
import jax.numpy as jnp
from jax.experimental import pallas as pl
from jax.experimental.pallas import tpu as pltpu


def _fused_body(num_res, x_ref, w1_ref, wres_ref, o_ref):
    xb = x_ref[...].astype(jnp.bfloat16)
    y = jnp.maximum(
        jnp.dot(xb, w1_ref[...], preferred_element_type=jnp.float32), 0.0)
    for r in range(num_res):
        h = jnp.maximum(
            jnp.dot(y.astype(jnp.bfloat16), wres_ref[2 * r],
                    preferred_element_type=jnp.float32), 0.0)
        t = jnp.maximum(
            jnp.dot(h.astype(jnp.bfloat16), wres_ref[2 * r + 1],
                    preferred_element_type=jnp.float32), 0.0)
        y = y + t
    o_ref[...] = y.astype(o_ref.dtype)


def _row_tile(n):
    for tm in (1024, 512, 256, 128, 64, 32, 16, 8):
        if n >= 4 * tm:
            return tm
    return 8


@jax.jit
def kernel(x, w1, wres):
    n, c = x.shape
    d = w1.shape[1]
    num_res = wres.shape[0] // 2
    out_dtype = x.dtype

    tm = _row_tile(n)
    grid = (pl.cdiv(n, tm),)

    def wspec(shape, index_map):
        return pl.BlockSpec(shape, index_map, pipeline_mode=pl.Buffered(1))

    return pl.pallas_call(
        functools.partial(_fused_body, num_res),
        out_shape=jax.ShapeDtypeStruct((n, d), out_dtype),
        grid=grid,
        in_specs=[
            pl.BlockSpec((tm, c), lambda i: (i, 0)),
            wspec((c, d), lambda i: (0, 0)),
            wspec((2 * num_res, d, d), lambda i: (0, 0, 0)),
        ],
        out_specs=pl.BlockSpec((tm, d), lambda i: (i, 0)),
        compiler_params=pltpu.CompilerParams(
            dimension_semantics=("parallel",)),
    )(x, w1, wres)
